# baseline (speedup 1.0000x reference)
import functools

import jax
import jax.numpy as jnp
from jax import lax
from jax.experimental import pallas as pl
from jax.experimental.pallas import tpu as pltpu
from jax.experimental.pallas import tpu_sc as plsc

N = 10000
E = 320000
H = 128

NP = 10240
ROWS_PER_TILE = NP // 16
ECHUNK = 128
EP_ROWS = 2560
EROWS_PER_TILE = EP_ROWS // 32
HALF = EROWS_PER_TILE // 2
HEAVY_PT = 128
LIGHT_PT = 32
HEAVY_CORE = 0
ACC_ROWS = 10112
ACC_RPT = ACC_ROWS // 16
DUMMY = 10050
BM = 512
GRID = NP // BM


def _deg_body(dst2d, zdeg, degp, acc2, dst_t, e0_buf, sem):
    c = lax.axis_index("c")
    s = lax.axis_index("s")
    row_base = (c * 16 + s) * EROWS_PER_TILE

    pltpu.sync_copy(zdeg, acc2.at[pl.ds(s * ACC_RPT, ACC_RPT)])

    e0 = jnp.where(lax.iota(jnp.int32, 16) == 0, 1.0, 0.0)
    zv = jnp.zeros((16,), jnp.float32)

    def fill(i, carry):
        def fill_col(cc, carry2):
            e0_buf[i, pl.ds(cc * 16, 16)] = jnp.where(cc == 0, e0, zv)
            return carry2
        lax.fori_loop(0, H // 16, fill_col, 0)
        return carry
    lax.fori_loop(0, ECHUNK, fill, 0)

    pltpu.sync_copy(dst2d.at[pl.ds(row_base, EROWS_PER_TILE)], dst_t)
    plsc.subcore_barrier()

    def scat(j, carry):
        pltpu.sync_copy(e0_buf, acc2.at[dst_t.at[j]], add=True)
        return carry
    lax.fori_loop(0, EROWS_PER_TILE, scat, 0)

    plsc.subcore_barrier()

    pltpu.sync_copy(acc2.at[pl.ds(s * ACC_RPT, ACC_RPT)],
                    degp.at[c, pl.ds(s * ACC_RPT, ACC_RPT)])


def _prop_pipeline(u_hbm, acc, src2d, dst2d, src_t, dst_t, rows0, rows1,
                   sem0, sem1, base, rows_pt, n_pass):
    half_pt = rows_pt // n_pass

    def half(hb, carry):
        rb = base + hb * half_pt
        pltpu.sync_copy(src2d.at[pl.ds(rb, half_pt)],
                        src_t.at[pl.ds(0, half_pt)])
        pltpu.sync_copy(dst2d.at[pl.ds(rb, half_pt)],
                        dst_t.at[pl.ds(0, half_pt)])
        pltpu.async_copy(u_hbm.at[src_t.at[0]], rows0, sem0)

        def pair(k, carry2):
            j = 2 * k
            pltpu.async_copy(u_hbm.at[src_t.at[j + 1]], rows1, sem1)
            pltpu.make_async_copy(u_hbm.at[src_t.at[j]], rows0, sem0).wait()
            pltpu.sync_copy(rows0, acc.at[dst_t.at[j]], add=True)
            pltpu.async_copy(u_hbm.at[src_t.at[j + 2]], rows0, sem0)
            pltpu.make_async_copy(u_hbm.at[src_t.at[j + 1]], rows1, sem1).wait()
            pltpu.sync_copy(rows1, acc.at[dst_t.at[j + 1]], add=True)
            return carry2
        lax.fori_loop(0, half_pt // 2 - 1, pair, 0)

        last = half_pt - 1
        pltpu.async_copy(u_hbm.at[src_t.at[last]], rows1, sem1)
        pltpu.make_async_copy(u_hbm.at[src_t.at[last - 1]], rows0, sem0).wait()
        pltpu.sync_copy(rows0, acc.at[dst_t.at[last - 1]], add=True)
        pltpu.make_async_copy(u_hbm.at[src_t.at[last]], rows1, sem1).wait()
        pltpu.sync_copy(rows1, acc.at[dst_t.at[last]], add=True)
        return carry
    lax.fori_loop(0, n_pass, half, 0)


def _prop_body(u_hbm, src2d, dst2d, zacc, out_hbm,
               acc, src_t, dst_t, rows0, rows1, sem0, sem1):
    c = lax.axis_index("c")
    s = lax.axis_index("s")

    pltpu.sync_copy(zacc, acc.at[pl.ds(s * ACC_RPT, ACC_RPT)])
    plsc.subcore_barrier()

    @pl.when(c == 0)
    def _core0():
        _prop_pipeline(u_hbm, acc, src2d, dst2d, src_t, dst_t, rows0, rows1,
                       sem0, sem1, s * (EP_ROWS // 16), EP_ROWS // 16, 4)

    plsc.subcore_barrier()
    pltpu.sync_copy(acc.at[pl.ds(s * ACC_RPT, ACC_RPT)],
                    out_hbm.at[c, pl.ds(s * ACC_RPT, ACC_RPT)])


@functools.lru_cache(maxsize=None)
def _sc_calls():
    mesh = plsc.VectorSubcoreMesh(core_axis_name="c", subcore_axis_name="s")
    deg_call = pl.kernel(
        _deg_body,
        out_type=jax.ShapeDtypeStruct((2, NP, H), jnp.float32),
        mesh=mesh,
        scratch_types=[
            pltpu.VMEM_SHARED((ACC_ROWS, H), jnp.float32),
            pltpu.VMEM((EROWS_PER_TILE, ECHUNK), jnp.int32),
            pltpu.VMEM((ECHUNK, H), jnp.float32),
            pltpu.SemaphoreType.DMA,
        ],
    )
    prop_call = pl.kernel(
        _prop_body,
        out_type=jax.ShapeDtypeStruct((2, NP, H), jnp.float32),
        mesh=mesh,
        scratch_types=[
            pltpu.VMEM_SHARED((ACC_ROWS, H), jnp.float32),
            pltpu.VMEM((EP_ROWS // 64, ECHUNK), jnp.int32),
            pltpu.VMEM((EP_ROWS // 64, ECHUNK), jnp.int32),
            pltpu.VMEM((ECHUNK, H), jnp.float32),
            pltpu.VMEM((ECHUNK, H), jnp.float32),
            pltpu.SemaphoreType.DMA,
            pltpu.SemaphoreType.DMA,
        ],
    )
    return deg_call, prop_call


def _dinv(dp0, dp1):
    return lax.rsqrt(dp0[...][:, 0:1] + dp1[...][:, 0:1] + 1.0)


def _tca_body(xc, wc, benc, w0, dp0, dp1, u_ref):
    h = jnp.dot(xc[...], wc[...], preferred_element_type=jnp.float32, precision=lax.Precision.HIGHEST) + benc[...]
    z = jnp.dot(h, w0[...], preferred_element_type=jnp.float32, precision=lax.Precision.HIGHEST)
    u_ref[...] = _dinv(dp0, dp1) * z


def _tcb_body(s0, s1, u, dp0, dp1, b, y_ref, cs_ref, cq_ref):
    i = pl.program_id(0)
    y = _dinv(dp0, dp1) * (s0[...] + s1[...] + u[...]) + b[...]
    rows = lax.broadcasted_iota(jnp.int32, (BM, 1), 0) + i * BM
    y = jnp.where(rows < N, y, 0.0)
    y_ref[...] = y

    @pl.when(i == 0)
    def _init():
        cs_ref[...] = jnp.zeros_like(cs_ref)
        cq_ref[...] = jnp.zeros_like(cq_ref)

    cs_ref[...] += jnp.sum(y, axis=0, keepdims=True)
    cq_ref[...] += jnp.sum(y * y, axis=0, keepdims=True)


def _tcc_body(y, cs, cq, g, be, w, dp0, dp1, u_ref):
    mu = cs[...] * (1.0 / N)
    var = cq[...] * (1.0 / N) - mu * mu
    rstd = lax.rsqrt(var + 1e-5)
    h = jnp.maximum((y[...] - mu) * rstd * g[...] + be[...], 0.0)
    z = jnp.dot(h, w[...], preferred_element_type=jnp.float32, precision=lax.Precision.HIGHEST)
    u_ref[...] = _dinv(dp0, dp1) * z


def _tcd_body(s0, s1, u, dp0, dp1, b2, wdec, bdec, o_ref):
    t = _dinv(dp0, dp1) * (s0[...] + s1[...] + u[...]) + b2[...]
    o_ref[...] = jnp.dot(t, wdec[...], preferred_element_type=jnp.float32, precision=lax.Precision.HIGHEST) + bdec[...]


def _rows_spec():
    return pl.BlockSpec((BM, H), lambda i: (i, 0))


def _col_spec():
    return pl.BlockSpec((BM, H), lambda i: (i, 0))


def _full_spec(shape):
    return pl.BlockSpec(shape, lambda i: tuple(0 for _ in shape))


_tca_call = pl.pallas_call(
    _tca_body,
    grid=(GRID,),
    in_specs=[
        pl.BlockSpec((BM, 256), lambda i: (i, 0)),
        _full_spec((256, H)),
        _full_spec((1, H)),
        _full_spec((H, H)),
        _col_spec(),
        _col_spec(),
    ],
    out_specs=_rows_spec(),
    out_shape=jax.ShapeDtypeStruct((NP, H), jnp.float32),
)

_tcb_call = pl.pallas_call(
    _tcb_body,
    grid=(GRID,),
    in_specs=[
        _rows_spec(),
        _rows_spec(),
        _rows_spec(),
        _col_spec(),
        _col_spec(),
        _full_spec((1, H)),
    ],
    out_specs=[_rows_spec(), _full_spec((1, H)), _full_spec((1, H))],
    out_shape=[
        jax.ShapeDtypeStruct((NP, H), jnp.float32),
        jax.ShapeDtypeStruct((1, H), jnp.float32),
        jax.ShapeDtypeStruct((1, H), jnp.float32),
    ],
)

_tcc_call = pl.pallas_call(
    _tcc_body,
    grid=(GRID,),
    in_specs=[
        _rows_spec(),
        _full_spec((1, H)),
        _full_spec((1, H)),
        _full_spec((1, H)),
        _full_spec((1, H)),
        _full_spec((H, H)),
        _col_spec(),
        _col_spec(),
    ],
    out_specs=_rows_spec(),
    out_shape=jax.ShapeDtypeStruct((NP, H), jnp.float32),
)

_tcd_call = pl.pallas_call(
    _tcd_body,
    grid=(GRID,),
    in_specs=[
        _rows_spec(),
        _rows_spec(),
        _rows_spec(),
        _col_spec(),
        _col_spec(),
        _full_spec((1, H)),
        _full_spec((H, H)),
        _full_spec((1, H)),
    ],
    out_specs=_rows_spec(),
    out_shape=jax.ShapeDtypeStruct((NP, H), jnp.float32),
)


def kernel(x, pe, edge_index, W_enc, b_enc, W0, b0, g0, be0, W1, b1, g1, be1,
           W2, b2, W_dec, b_dec):
    f32 = jnp.float32
    src = edge_index[0]
    dst = edge_index[1]

    pad_e = EP_ROWS * ECHUNK - E
    pad_dst = N + (jnp.arange(pad_e, dtype=jnp.int32) % (ACC_ROWS - N))
    src_p = jnp.concatenate(
        [src, jnp.zeros((pad_e,), jnp.int32)]).reshape(EP_ROWS, ECHUNK)
    dst_p = jnp.concatenate([dst, pad_dst]).reshape(EP_ROWS, ECHUNK)

    xc = jnp.zeros((NP, 256), f32).at[:N, :128].set(x).at[:N, 128:132].set(pe)
    wc = jnp.zeros((256, H), f32).at[:132, :].set(W_enc)

    zdeg = jnp.zeros((ACC_RPT, H), f32)
    zacc = jnp.zeros((ACC_RPT, H), f32)

    r1 = lambda v: v.reshape(1, H)

    _deg_call, _prop_call = _sc_calls()
    degp = _deg_call(dst_p, zdeg)
    dp0 = degp[0]
    dp1 = degp[1]

    u0 = _tca_call(xc, wc, r1(b_enc), W0, dp0, dp1)
    s = _prop_call(u0, src_p, dst_p, zacc)
    y0, cs0, cq0 = _tcb_call(s[0], s[1], u0, dp0, dp1, r1(b0))

    u1 = _tcc_call(y0, cs0, cq0, r1(g0), r1(be0), W1, dp0, dp1)
    s = _prop_call(u1, src_p, dst_p, zacc)
    y1, cs1, cq1 = _tcb_call(s[0], s[1], u1, dp0, dp1, r1(b1))

    u2 = _tcc_call(y1, cs1, cq1, r1(g1), r1(be1), W2, dp0, dp1)
    s = _prop_call(u2, src_p, dst_p, zacc)
    out = _tcd_call(s[0], s[1], u2, dp0, dp1, r1(b2), W_dec, r1(b_dec))

    return out[:N]


# speedup vs baseline: 1.1335x; 1.1335x over previous
import functools

import jax
import jax.numpy as jnp
from jax import lax
from jax.experimental import pallas as pl
from jax.experimental.pallas import tpu as pltpu
from jax.experimental.pallas import tpu_sc as plsc

N = 10000
E = 320000
H = 128

NP = 10240
ROWS_PER_TILE = NP // 16
ECHUNK = 128
EP_ROWS = 2560
EROWS_PER_TILE = EP_ROWS // 32
HALF = EROWS_PER_TILE // 2
HEAVY_PT = 128
LIGHT_PT = 32
HEAVY_CORE = 0
ACC_ROWS = 10112
ACC_RPT = ACC_ROWS // 16
DUMMY = 10050
BM = 512
GRID = NP // BM


def _deg_body(dst2d, zdeg, degp, acc2, dst_t, e0_buf, sem):
    c = lax.axis_index("c")
    s = lax.axis_index("s")
    row_base = (c * 16 + s) * EROWS_PER_TILE

    pltpu.sync_copy(zdeg, acc2.at[pl.ds(s * ACC_RPT, ACC_RPT)])

    e0 = jnp.where(lax.iota(jnp.int32, 16) == 0, 1.0, 0.0)
    zv = jnp.zeros((16,), jnp.float32)

    def fill(i, carry):
        def fill_col(cc, carry2):
            e0_buf[i, pl.ds(cc * 16, 16)] = jnp.where(cc == 0, e0, zv)
            return carry2
        lax.fori_loop(0, H // 16, fill_col, 0)
        return carry
    lax.fori_loop(0, ECHUNK, fill, 0)

    pltpu.sync_copy(dst2d.at[pl.ds(row_base, EROWS_PER_TILE)], dst_t)
    plsc.subcore_barrier()

    def scat(j, carry):
        pltpu.sync_copy(e0_buf, acc2.at[dst_t.at[j]], add=True)
        return carry
    lax.fori_loop(0, EROWS_PER_TILE, scat, 0)

    plsc.subcore_barrier()

    pltpu.sync_copy(acc2.at[pl.ds(s * ACC_RPT, ACC_RPT)],
                    degp.at[c, pl.ds(s * ACC_RPT, ACC_RPT)])


def _prop_pipeline(u_hbm, acc, src2d, dst2d, src_t, dst_t, rows0, rows1,
                   sem0, sem1, base, rows_pt, n_pass):
    half_pt = rows_pt // n_pass

    def half(hb, carry):
        rb = base + hb * half_pt
        pltpu.sync_copy(src2d.at[pl.ds(rb, half_pt)],
                        src_t.at[pl.ds(0, half_pt)])
        pltpu.sync_copy(dst2d.at[pl.ds(rb, half_pt)],
                        dst_t.at[pl.ds(0, half_pt)])
        pltpu.async_copy(u_hbm.at[src_t.at[0]], rows0, sem0)

        def pair(k, carry2):
            j = 2 * k
            pltpu.async_copy(u_hbm.at[src_t.at[j + 1]], rows1, sem1)
            pltpu.make_async_copy(u_hbm.at[src_t.at[j]], rows0, sem0).wait()
            pltpu.sync_copy(rows0, acc.at[dst_t.at[j]], add=True)
            pltpu.async_copy(u_hbm.at[src_t.at[j + 2]], rows0, sem0)
            pltpu.make_async_copy(u_hbm.at[src_t.at[j + 1]], rows1, sem1).wait()
            pltpu.sync_copy(rows1, acc.at[dst_t.at[j + 1]], add=True)
            return carry2
        lax.fori_loop(0, half_pt // 2 - 1, pair, 0)

        last = half_pt - 1
        pltpu.async_copy(u_hbm.at[src_t.at[last]], rows1, sem1)
        pltpu.make_async_copy(u_hbm.at[src_t.at[last - 1]], rows0, sem0).wait()
        pltpu.sync_copy(rows0, acc.at[dst_t.at[last - 1]], add=True)
        pltpu.make_async_copy(u_hbm.at[src_t.at[last]], rows1, sem1).wait()
        pltpu.sync_copy(rows1, acc.at[dst_t.at[last]], add=True)
        return carry
    lax.fori_loop(0, n_pass, half, 0)


def _prop_body(u_hbm, src2d, dst2d, zacc, out_hbm,
               acc, src_t, dst_t, rows0, rows1, sem0, sem1):
    c = lax.axis_index("c")
    s = lax.axis_index("s")

    pltpu.sync_copy(zacc, acc.at[pl.ds(s * ACC_RPT, ACC_RPT)])
    plsc.subcore_barrier()

    rows_c0 = HEAVY_PT if HEAVY_CORE == 0 else LIGHT_PT
    rows_c1 = HEAVY_PT if HEAVY_CORE == 1 else LIGHT_PT

    @pl.when(c == 0)
    def _core0():
        _prop_pipeline(u_hbm, acc, src2d, dst2d, src_t, dst_t, rows0, rows1,
                       sem0, sem1, s * rows_c0, rows_c0, 2)

    @pl.when(c == 1)
    def _core1():
        _prop_pipeline(u_hbm, acc, src2d, dst2d, src_t, dst_t, rows0, rows1,
                       sem0, sem1, 16 * rows_c0 + s * rows_c1, rows_c1, 2)

    plsc.subcore_barrier()
    pltpu.sync_copy(acc.at[pl.ds(s * ACC_RPT, ACC_RPT)],
                    out_hbm.at[c, pl.ds(s * ACC_RPT, ACC_RPT)])


@functools.lru_cache(maxsize=None)
def _sc_calls():
    mesh = plsc.VectorSubcoreMesh(core_axis_name="c", subcore_axis_name="s")
    deg_call = pl.kernel(
        _deg_body,
        out_type=jax.ShapeDtypeStruct((2, NP, H), jnp.float32),
        mesh=mesh,
        scratch_types=[
            pltpu.VMEM_SHARED((ACC_ROWS, H), jnp.float32),
            pltpu.VMEM((EROWS_PER_TILE, ECHUNK), jnp.int32),
            pltpu.VMEM((ECHUNK, H), jnp.float32),
            pltpu.SemaphoreType.DMA,
        ],
    )
    prop_call = pl.kernel(
        _prop_body,
        out_type=jax.ShapeDtypeStruct((2, NP, H), jnp.float32),
        mesh=mesh,
        scratch_types=[
            pltpu.VMEM_SHARED((ACC_ROWS, H), jnp.float32),
            pltpu.VMEM((HEAVY_PT // 2, ECHUNK), jnp.int32),
            pltpu.VMEM((HEAVY_PT // 2, ECHUNK), jnp.int32),
            pltpu.VMEM((ECHUNK, H), jnp.float32),
            pltpu.VMEM((ECHUNK, H), jnp.float32),
            pltpu.SemaphoreType.DMA,
            pltpu.SemaphoreType.DMA,
        ],
    )
    return deg_call, prop_call


def _dinv(dp0, dp1):
    return lax.rsqrt(dp0[...][:, 0:1] + dp1[...][:, 0:1] + 1.0)


def _tca_body(xc, wc, benc, w0, dp0, dp1, u_ref):
    h = jnp.dot(xc[...], wc[...], preferred_element_type=jnp.float32, precision=lax.Precision.HIGHEST) + benc[...]
    z = jnp.dot(h, w0[...], preferred_element_type=jnp.float32, precision=lax.Precision.HIGHEST)
    u_ref[...] = _dinv(dp0, dp1) * z


def _tcb_body(s0, s1, u, dp0, dp1, b, y_ref, cs_ref, cq_ref):
    i = pl.program_id(0)
    y = _dinv(dp0, dp1) * (s0[...] + s1[...] + u[...]) + b[...]
    rows = lax.broadcasted_iota(jnp.int32, (BM, 1), 0) + i * BM
    y = jnp.where(rows < N, y, 0.0)
    y_ref[...] = y

    @pl.when(i == 0)
    def _init():
        cs_ref[...] = jnp.zeros_like(cs_ref)
        cq_ref[...] = jnp.zeros_like(cq_ref)

    cs_ref[...] += jnp.sum(y, axis=0, keepdims=True)
    cq_ref[...] += jnp.sum(y * y, axis=0, keepdims=True)


def _tcc_body(y, cs, cq, g, be, w, dp0, dp1, u_ref):
    mu = cs[...] * (1.0 / N)
    var = cq[...] * (1.0 / N) - mu * mu
    rstd = lax.rsqrt(var + 1e-5)
    h = jnp.maximum((y[...] - mu) * rstd * g[...] + be[...], 0.0)
    z = jnp.dot(h, w[...], preferred_element_type=jnp.float32, precision=lax.Precision.HIGHEST)
    u_ref[...] = _dinv(dp0, dp1) * z


def _tcd_body(s0, s1, u, dp0, dp1, b2, wdec, bdec, o_ref):
    t = _dinv(dp0, dp1) * (s0[...] + s1[...] + u[...]) + b2[...]
    o_ref[...] = jnp.dot(t, wdec[...], preferred_element_type=jnp.float32, precision=lax.Precision.HIGHEST) + bdec[...]


def _rows_spec():
    return pl.BlockSpec((BM, H), lambda i: (i, 0))


def _col_spec():
    return pl.BlockSpec((BM, H), lambda i: (i, 0))


def _full_spec(shape):
    return pl.BlockSpec(shape, lambda i: tuple(0 for _ in shape))


_tca_call = pl.pallas_call(
    _tca_body,
    grid=(GRID,),
    in_specs=[
        pl.BlockSpec((BM, 256), lambda i: (i, 0)),
        _full_spec((256, H)),
        _full_spec((1, H)),
        _full_spec((H, H)),
        _col_spec(),
        _col_spec(),
    ],
    out_specs=_rows_spec(),
    out_shape=jax.ShapeDtypeStruct((NP, H), jnp.float32),
)

_tcb_call = pl.pallas_call(
    _tcb_body,
    grid=(GRID,),
    in_specs=[
        _rows_spec(),
        _rows_spec(),
        _rows_spec(),
        _col_spec(),
        _col_spec(),
        _full_spec((1, H)),
    ],
    out_specs=[_rows_spec(), _full_spec((1, H)), _full_spec((1, H))],
    out_shape=[
        jax.ShapeDtypeStruct((NP, H), jnp.float32),
        jax.ShapeDtypeStruct((1, H), jnp.float32),
        jax.ShapeDtypeStruct((1, H), jnp.float32),
    ],
)

_tcc_call = pl.pallas_call(
    _tcc_body,
    grid=(GRID,),
    in_specs=[
        _rows_spec(),
        _full_spec((1, H)),
        _full_spec((1, H)),
        _full_spec((1, H)),
        _full_spec((1, H)),
        _full_spec((H, H)),
        _col_spec(),
        _col_spec(),
    ],
    out_specs=_rows_spec(),
    out_shape=jax.ShapeDtypeStruct((NP, H), jnp.float32),
)

_tcd_call = pl.pallas_call(
    _tcd_body,
    grid=(GRID,),
    in_specs=[
        _rows_spec(),
        _rows_spec(),
        _rows_spec(),
        _col_spec(),
        _col_spec(),
        _full_spec((1, H)),
        _full_spec((H, H)),
        _full_spec((1, H)),
    ],
    out_specs=_rows_spec(),
    out_shape=jax.ShapeDtypeStruct((NP, H), jnp.float32),
)


def kernel(x, pe, edge_index, W_enc, b_enc, W0, b0, g0, be0, W1, b1, g1, be1,
           W2, b2, W_dec, b_dec):
    f32 = jnp.float32
    src = edge_index[0]
    dst = edge_index[1]

    pad_e = EP_ROWS * ECHUNK - E
    pad_dst = N + (jnp.arange(pad_e, dtype=jnp.int32) % (ACC_ROWS - N))
    src_p = jnp.concatenate(
        [src, jnp.zeros((pad_e,), jnp.int32)]).reshape(EP_ROWS, ECHUNK)
    dst_p = jnp.concatenate([dst, pad_dst]).reshape(EP_ROWS, ECHUNK)

    xc = jnp.zeros((NP, 256), f32).at[:N, :128].set(x).at[:N, 128:132].set(pe)
    wc = jnp.zeros((256, H), f32).at[:132, :].set(W_enc)

    zdeg = jnp.zeros((ACC_RPT, H), f32)
    zacc = jnp.zeros((ACC_RPT, H), f32)

    r1 = lambda v: v.reshape(1, H)

    _deg_call, _prop_call = _sc_calls()
    degp = _deg_call(dst_p, zdeg)
    dp0 = degp[0]
    dp1 = degp[1]

    u0 = _tca_call(xc, wc, r1(b_enc), W0, dp0, dp1)
    s = _prop_call(u0, src_p, dst_p, zacc)
    y0, cs0, cq0 = _tcb_call(s[0], s[1], u0, dp0, dp1, r1(b0))

    u1 = _tcc_call(y0, cs0, cq0, r1(g0), r1(be0), W1, dp0, dp1)
    s = _prop_call(u1, src_p, dst_p, zacc)
    y1, cs1, cq1 = _tcb_call(s[0], s[1], u1, dp0, dp1, r1(b1))

    u2 = _tcc_call(y1, cs1, cq1, r1(g1), r1(be1), W2, dp0, dp1)
    s = _prop_call(u2, src_p, dst_p, zacc)
    out = _tcd_call(s[0], s[1], u2, dp0, dp1, r1(b2), W_dec, r1(b_dec))

    return out[:N]
